# baseline (speedup 1.0000x reference)
import functools

import jax
import jax.numpy as jnp
from jax import lax
from jax.experimental import pallas as pl
from jax.experimental.pallas import tpu as pltpu
from jax.experimental.pallas import tpu_sc as plsc

_B, _N, _D, _H, _V, _PV, _A = 8, 2048, 16, 128, 10000, 32, 16
_BN = _B * _N
_PAD = 0
_F32 = jnp.float32
_BF16 = jnp.bfloat16


def _bdot(a, b):
    return jnp.dot(a.astype(_BF16), b.astype(_BF16),
                   preferred_element_type=_F32)


def _sc_gather(table, idx3):
    info = plsc.get_sparse_core_info()
    nw = info.num_cores * info.num_subcores
    kch = idx3.shape[1]

    mesh = plsc.VectorSubcoreMesh(core_axis_name="c", subcore_axis_name="s")

    @functools.partial(
        pl.kernel,
        out_type=jax.ShapeDtypeStruct((nw, kch, 128, _H), jnp.float32),
        mesh=mesh,
        scratch_types=[
            pltpu.VMEM((kch, 128), jnp.int32),
            pltpu.VMEM((kch, 128, _H), jnp.float32),
            pltpu.SemaphoreType.DMA,
        ],
    )
    def k(table_hbm, idx_hbm, out_hbm, idx_v, rows_v, sem):
        wid = lax.axis_index("s") * info.num_cores + lax.axis_index("c")
        pltpu.sync_copy(idx_hbm.at[wid], idx_v)
        copies = [
            pltpu.async_copy(table_hbm.at[idx_v.at[j]], rows_v.at[j], sem)
            for j in range(kch)
        ]
        for c in copies:
            c.wait()
        pltpu.sync_copy(rows_v, out_hbm.at[wid])

    return k(table, idx3)


def _rnn_body(pos_ref, pet_ref, wih1_ref, b1_ref, whh1_ref, w2_ref, b2_ref,
              out_ref):
    pos = pos_ref[...]
    c = pos.shape[0]
    msk = pos != _PAD
    lengths = jnp.sum(msk.astype(jnp.int32), axis=1, keepdims=True)
    last_idx = jnp.clip(lengths - 1, 0, _D - 1)
    t1 = jnp.dot(pet_ref[...], wih1_ref[...], preferred_element_type=_F32)
    t1 = t1 + b1_ref[...]
    b2v = b2_ref[...]
    iota_pv = lax.broadcasted_iota(jnp.int32, (c, _PV), 1)
    h1 = jnp.zeros((c, _H), _F32)
    h2 = jnp.zeros((c, _H), _F32)
    fin = jnp.zeros((c, _H), _F32)
    for t in range(_D):
        pos_t = lax.slice_in_dim(pos, t, t + 1, axis=1)
        m_t = pos_t != _PAD
        oh = (pos_t == iota_pv).astype(_F32)
        x1 = _bdot(oh, t1)
        a1 = x1 + _bdot(h1, whh1_ref[...])
        h1 = jnp.where(m_t, jnp.tanh(a1), h1)
        cat = jnp.concatenate([h1, h2], axis=1)
        a2 = _bdot(cat, w2_ref[...]) + b2v
        h2 = jnp.where(m_t, jnp.tanh(a2), h2)
        fin = jnp.where(last_idx == t, h2, fin)
    out_ref[...] = fin


def _enc_body(pf_ref, nf_ref, par_ref, tn_ref, wc_ref, bc_ref, wt1_ref,
              bt1_ref, wt2_ref, bt2_ref, wa_ref, ba_ref, out_ref):
    cat = jnp.concatenate([pf_ref[...], nf_ref[...]], axis=1)
    h = jnp.tanh(_bdot(cat, wc_ref[...]) + bc_ref[...])
    mf = (tn_ref[...] != _PAD).astype(_F32)
    par = par_ref[...]
    iota_n = lax.broadcasted_iota(jnp.int32, (_N, _N), 1)
    p = (par == iota_n).astype(_BF16)
    ph = _bdot(p, h)
    h = jnp.tanh(_bdot(jnp.concatenate([h, ph], axis=1), wt1_ref[...])
                 + bt1_ref[...]) * mf
    ph = _bdot(p, h)
    h = jnp.tanh(_bdot(jnp.concatenate([h, ph], axis=1), wt2_ref[...])
                 + bt2_ref[...]) * mf
    out_ref[...] = _bdot(h, wa_ref[...]) + ba_ref[...]


def _const_spec(shape):
    return pl.BlockSpec(shape, lambda i: (0,) * len(shape))


def kernel(tree_nodes, node_pos, node_parents, node_emb, pos_emb_table,
           Wih1, Whh1, b1, Wih2, Whh2, b2, Wc, bc,
           Wx1, Wp1, bt1, Wx2, Wp2, bt2, Wa, ba):
    tn = tree_nodes.astype(jnp.int32)
    pos2 = node_pos.astype(jnp.int32).reshape(_BN, _D)
    par2 = node_parents.astype(jnp.int32).reshape(_BN, 1)
    tn2 = tn.reshape(_BN, 1)

    idx3 = tn.reshape(32, _BN // (32 * 128), 128)
    nf = _sc_gather(node_emb, idx3).reshape(_BN, _H)

    w2 = jnp.concatenate([Wih2, Whh2], axis=0)
    wt1 = jnp.concatenate([Wx1, Wp1], axis=0)
    wt2 = jnp.concatenate([Wx2, Wp2], axis=0)
    b1r = b1.reshape(1, _H)
    b2r = b2.reshape(1, _H)
    bcr = bc.reshape(1, _H)
    bt1r = bt1.reshape(1, _H)
    bt2r = bt2.reshape(1, _H)
    bar = ba.reshape(1, _A)

    grid = (_B,)
    row_spec = pl.BlockSpec((_N, _H), lambda i: (i, 0))

    pf = pl.pallas_call(
        _rnn_body,
        grid=grid,
        in_specs=[
            pl.BlockSpec((_N, _D), lambda i: (i, 0)),
            _const_spec((_PV, _H)),
            _const_spec((_H, _H)),
            _const_spec((1, _H)),
            _const_spec((_H, _H)),
            _const_spec((2 * _H, _H)),
            _const_spec((1, _H)),
        ],
        out_specs=row_spec,
        out_shape=jax.ShapeDtypeStruct((_BN, _H), _F32),
    )(pos2, pos_emb_table, Wih1, b1r, Whh1, w2, b2r)

    out = pl.pallas_call(
        _enc_body,
        grid=grid,
        in_specs=[
            row_spec,
            row_spec,
            pl.BlockSpec((_N, 1), lambda i: (i, 0)),
            pl.BlockSpec((_N, 1), lambda i: (i, 0)),
            _const_spec((2 * _H, _H)),
            _const_spec((1, _H)),
            _const_spec((2 * _H, _H)),
            _const_spec((1, _H)),
            _const_spec((2 * _H, _H)),
            _const_spec((1, _H)),
            _const_spec((_H, _A)),
            _const_spec((1, _A)),
        ],
        out_specs=pl.BlockSpec((_N, _A), lambda i: (i, 0)),
        out_shape=jax.ShapeDtypeStruct((_BN, _A), _F32),
    )(pf, nf, par2, tn2, Wc, bcr, wt1, bt1r, wt2, bt2r, Wa, bar)

    node_logits = out.reshape(_B, _N, _A)
    node_mask = tree_nodes != _PAD
    return node_logits, node_mask


# speedup vs baseline: 1.1522x; 1.1522x over previous
import functools

import jax
import jax.numpy as jnp
from jax import lax
from jax.experimental import pallas as pl
from jax.experimental.pallas import tpu as pltpu
from jax.experimental.pallas import tpu_sc as plsc

_B, _N, _D, _H, _V, _PV, _A = 8, 2048, 16, 128, 10000, 32, 16
_BN = _B * _N
_PAD = 0
_F32 = jnp.float32
_BF16 = jnp.bfloat16


def _bdot(a, b):
    return jnp.dot(a.astype(_BF16), b.astype(_BF16),
                   preferred_element_type=_F32)


def _sc_gather(table, idx3):
    info = plsc.get_sparse_core_info()
    nw = info.num_cores * info.num_subcores
    kch = idx3.shape[1]

    mesh = plsc.VectorSubcoreMesh(core_axis_name="c", subcore_axis_name="s")

    @functools.partial(
        pl.kernel,
        out_type=jax.ShapeDtypeStruct((nw, kch, 128, _H), jnp.float32),
        mesh=mesh,
        scratch_types=[
            pltpu.VMEM((kch, 128), jnp.int32),
            pltpu.VMEM((kch, 128, _H), jnp.float32),
            pltpu.SemaphoreType.DMA,
        ],
    )
    def k(table_hbm, idx_hbm, out_hbm, idx_v, rows_v, sem):
        wid = lax.axis_index("s") * info.num_cores + lax.axis_index("c")
        pltpu.sync_copy(idx_hbm.at[wid], idx_v)
        copies = [
            pltpu.async_copy(table_hbm.at[idx_v.at[j]], rows_v.at[j], sem)
            for j in range(kch)
        ]
        for c in copies:
            c.wait()
        pltpu.sync_copy(rows_v, out_hbm.at[wid])

    return k(table, idx3)


def _gather_rows(table, idx):
    idx3 = idx.reshape(32, idx.shape[0] // (32 * 128), 128)
    return _sc_gather(table, idx3).reshape(idx.shape[0], _H)


def _rnn_body(pos_ref, pet_ref, wih1_ref, b1_ref, wa_ref, whh2_ref, b2_ref,
              out_ref):
    pos = pos_ref[...]
    c = pos.shape[0]
    msk = pos != _PAD
    lengths = jnp.sum(msk.astype(jnp.int32), axis=1, keepdims=True)
    last_idx = jnp.clip(lengths - 1, 0, _D - 1)
    t1 = jnp.dot(pet_ref[...], wih1_ref[...], preferred_element_type=_F32)
    t1 = t1 + b1_ref[...]
    b2v = b2_ref[...]
    iota_pv = lax.broadcasted_iota(jnp.int32, (c, _PV), 1)
    h1 = jnp.zeros((c, _H), _F32)
    h2 = jnp.zeros((c, _H), _F32)
    fin = jnp.zeros((c, _H), _F32)
    m_prev = None
    for t in range(_D + 1):
        u = _bdot(h1, wa_ref[...])
        if t < _D:
            pos_t = lax.slice_in_dim(pos, t, t + 1, axis=1)
            m_t = pos_t != _PAD
            oh = (pos_t == iota_pv).astype(_F32)
            x1 = _bdot(oh, t1)
        if t >= 1:
            v = _bdot(h2, whh2_ref[...])
            a2 = lax.slice_in_dim(u, _H, 2 * _H, axis=1) + v + b2v
            h2 = jnp.where(m_prev, jnp.tanh(a2), h2)
            fin = jnp.where(last_idx == t - 1, h2, fin)
        if t < _D:
            a1 = x1 + lax.slice_in_dim(u, 0, _H, axis=1)
            h1 = jnp.where(m_t, jnp.tanh(a1), h1)
            m_prev = m_t
    out_ref[...] = fin


def _composer_body(pf_ref, nf_ref, wc_ref, bc_ref, out_ref):
    cat = jnp.concatenate([pf_ref[...], nf_ref[...]], axis=1)
    out_ref[...] = jnp.tanh(_bdot(cat, wc_ref[...]) + bc_ref[...])


def _combine_body(h_ref, ph_ref, tn_ref, w_ref, b_ref, out_ref):
    mf = (tn_ref[...] != _PAD).astype(_F32)
    cat = jnp.concatenate([h_ref[...], ph_ref[...]], axis=1)
    out_ref[...] = jnp.tanh(_bdot(cat, w_ref[...]) + b_ref[...]) * mf


def _head_body(h_ref, ph_ref, tn_ref, w_ref, b_ref, wa_ref, ba_ref, out_ref):
    mf = (tn_ref[...] != _PAD).astype(_F32)
    cat = jnp.concatenate([h_ref[...], ph_ref[...]], axis=1)
    h = jnp.tanh(_bdot(cat, w_ref[...]) + b_ref[...]) * mf
    out_ref[...] = _bdot(h, wa_ref[...]) + ba_ref[...]


def _const_spec(shape):
    return pl.BlockSpec(shape, lambda i: (0,) * len(shape))


def kernel(tree_nodes, node_pos, node_parents, node_emb, pos_emb_table,
           Wih1, Whh1, b1, Wih2, Whh2, b2, Wc, bc,
           Wx1, Wp1, bt1, Wx2, Wp2, bt2, Wa, ba):
    tn = tree_nodes.astype(jnp.int32)
    pos2 = node_pos.astype(jnp.int32).reshape(_BN, _D)
    tn2 = tn.reshape(_BN, 1)
    parg = (node_parents.astype(jnp.int32)
            + _N * jnp.arange(_B, dtype=jnp.int32)[:, None]).reshape(_BN)

    nf = _gather_rows(node_emb, tn.reshape(_BN))

    wa_rnn = jnp.concatenate([Whh1, Wih2], axis=1)
    wt1 = jnp.concatenate([Wx1, Wp1], axis=0)
    wt2 = jnp.concatenate([Wx2, Wp2], axis=0)
    b1r = b1.reshape(1, _H)
    b2r = b2.reshape(1, _H)
    bcr = bc.reshape(1, _H)
    bt1r = bt1.reshape(1, _H)
    bt2r = bt2.reshape(1, _H)
    bar = ba.reshape(1, _A)

    ca = _BN // 4
    pf = pl.pallas_call(
        _rnn_body,
        grid=(_BN // ca,),
        in_specs=[
            pl.BlockSpec((ca, _D), lambda i: (i, 0)),
            _const_spec((_PV, _H)),
            _const_spec((_H, _H)),
            _const_spec((1, _H)),
            _const_spec((_H, 2 * _H)),
            _const_spec((_H, _H)),
            _const_spec((1, _H)),
        ],
        out_specs=pl.BlockSpec((ca, _H), lambda i: (i, 0)),
        out_shape=jax.ShapeDtypeStruct((_BN, _H), _F32),
    )(pos2, pos_emb_table, Wih1, b1r, wa_rnn, Whh2, b2r)

    cb = _BN // 4
    row = pl.BlockSpec((cb, _H), lambda i: (i, 0))
    col = pl.BlockSpec((cb, 1), lambda i: (i, 0))

    h0 = pl.pallas_call(
        _composer_body,
        grid=(_BN // cb,),
        in_specs=[row, row, _const_spec((2 * _H, _H)), _const_spec((1, _H))],
        out_specs=row,
        out_shape=jax.ShapeDtypeStruct((_BN, _H), _F32),
    )(pf, nf, Wc, bcr)

    g1 = _gather_rows(h0, parg)

    h1m = pl.pallas_call(
        _combine_body,
        grid=(_BN // cb,),
        in_specs=[row, row, col, _const_spec((2 * _H, _H)),
                  _const_spec((1, _H))],
        out_specs=row,
        out_shape=jax.ShapeDtypeStruct((_BN, _H), _F32),
    )(h0, g1, tn2, wt1, bt1r)

    g2 = _gather_rows(h1m, parg)

    out = pl.pallas_call(
        _head_body,
        grid=(_BN // cb,),
        in_specs=[row, row, col, _const_spec((2 * _H, _H)),
                  _const_spec((1, _H)), _const_spec((_H, _A)),
                  _const_spec((1, _A))],
        out_specs=pl.BlockSpec((cb, _A), lambda i: (i, 0)),
        out_shape=jax.ShapeDtypeStruct((_BN, _A), _F32),
    )(h1m, g2, tn2, wt2, bt2r, Wa, bar)

    node_logits = out.reshape(_B, _N, _A)
    node_mask = tree_nodes != _PAD
    return node_logits, node_mask
